# Initial kernel scaffold; baseline (speedup 1.0000x reference)
import jax
import jax.numpy as jnp
from jax.experimental import pallas as pl


def kernel(table, input_tensor):
    raise NotImplementedError("write your pallas kernel here")


# speedup vs baseline: 2.4032x; 2.4032x over previous
import jax
import jax.numpy as jnp
from jax import lax
from jax.experimental import pallas as pl
from jax.experimental.pallas import tpu as pltpu
from jax.experimental.pallas import tpu_sc as plsc

_VOCAB = 100000
_EMB = 64
_B = 4096
_L = 200

_N = _B * _L
_NC = 2
_NS = 16
_NW = _NC * _NS
_CH = 512
_PER_W = _N // _NW
_STEPS = _PER_W // _CH

_BT = 8


def _sc_gather(table, flat_idx):
    mesh = plsc.VectorSubcoreMesh(core_axis_name="c", subcore_axis_name="s")

    @pl.kernel(
        out_type=jax.ShapeDtypeStruct((_N, _EMB), table.dtype),
        mesh=mesh,
        compiler_params=pltpu.CompilerParams(use_tc_tiling_on_sc=False),
        scratch_types=[
            pltpu.VMEM((_CH,), jnp.int32),
            pltpu.VMEM((_CH, _EMB), jnp.float32),
            pltpu.SemaphoreType.DMA,
        ],
    )
    def gather_kernel(table_hbm, idx_hbm, out_hbm, idx_v, rows_v, sem):
        wid = lax.axis_index("s") * _NC + lax.axis_index("c")

        @pl.loop(0, _STEPS)
        def _(step):
            base = wid * _PER_W + step * _CH
            pltpu.sync_copy(idx_hbm.at[pl.ds(base, _CH)], idx_v)
            pltpu.async_copy(table_hbm.at[idx_v], rows_v, sem).wait()
            pltpu.sync_copy(rows_v, out_hbm.at[pl.ds(base, _CH)])

    return gather_kernel(table, flat_idx)


def _tc_transpose(x):
    def body(x_ref, o_ref):
        o_ref[...] = jnp.transpose(x_ref[...], (0, 2, 1))

    return pl.pallas_call(
        body,
        grid=(_B // _BT,),
        in_specs=[pl.BlockSpec((_BT, _L, _EMB), lambda i: (i, 0, 0))],
        out_specs=pl.BlockSpec((_BT, _EMB, _L), lambda i: (i, 0, 0)),
        out_shape=jax.ShapeDtypeStruct((_B, _EMB, _L), x.dtype),
    )(x)


def kernel(table, input_tensor):
    flat_idx = input_tensor.reshape(_N)
    gathered = _sc_gather(table, flat_idx)
    return _tc_transpose(gathered.reshape(_B, _L, _EMB))
